# baseline (speedup 1.0000x reference)
import jax
import jax.numpy as jnp
from jax.experimental import pallas as pl
from jax.experimental.pallas import tpu as pltpu

NEG_INF = float("-inf")
BLK = 40
SUB = 1


def _nms_one(x):
    row = jax.lax.broadcasted_iota(jnp.int32, x.shape, 1)
    h = x.shape[1]
    interior = jnp.maximum(jnp.maximum(x[:, :, :-2], x[:, :, 1:-1]), x[:, :, 2:])
    first = jnp.maximum(x[:, :, 0:1], x[:, :, 1:2])
    last = jnp.maximum(x[:, :, -2:-1], x[:, :, -1:])
    rowmax = jnp.concatenate([first, interior, last], axis=2)
    up = jnp.where(row == 0, NEG_INF, pltpu.roll(rowmax, 1, 1))
    down = jnp.where(row == h - 1, NEG_INF, pltpu.roll(rowmax, h - 1, 1))
    hmax = jnp.maximum(jnp.maximum(up, rowmax), down)
    return jnp.where(hmax == x, x, 0.0)


def _nms_body(x_ref, o_ref):
    for s in range(BLK // SUB):
        x = x_ref[s * SUB : (s + 1) * SUB]
        o_ref[s * SUB : (s + 1) * SUB] = _nms_one(x)


def kernel(points):
    n, c, h, w = points.shape
    x = points.reshape(n * c, h, w)
    out = pl.pallas_call(
        _nms_body,
        grid=((n * c) // BLK,),
        in_specs=[pl.BlockSpec((BLK, h, w), lambda i: (i, 0, 0))],
        out_specs=pl.BlockSpec((BLK, h, w), lambda i: (i, 0, 0)),
        out_shape=jax.ShapeDtypeStruct((n * c, h, w), points.dtype),
        compiler_params=pltpu.CompilerParams(vmem_limit_bytes=128 * 1024 * 1024),
    )(x)
    return out.reshape(n, c, h, w)


# speedup vs baseline: 1.4019x; 1.4019x over previous
import jax
import jax.numpy as jnp
from jax.experimental import pallas as pl
from jax.experimental.pallas import tpu as pltpu

NEG_INF = float("-inf")
BLK = 40
SUB = 1


def _nms_one(x):
    row = jax.lax.broadcasted_iota(jnp.int32, x.shape, 1)
    h = x.shape[1]
    left = jnp.concatenate([jnp.full_like(x[:, :, :1], NEG_INF), x[:, :, :-1]], axis=2)
    right = jnp.concatenate([x[:, :, 1:], jnp.full_like(x[:, :, :1], NEG_INF)], axis=2)
    rowmax = jnp.maximum(jnp.maximum(left, x), right)
    up = jnp.where(row == 0, NEG_INF, pltpu.roll(rowmax, 1, 1))
    down = jnp.where(row == h - 1, NEG_INF, pltpu.roll(rowmax, h - 1, 1))
    hmax = jnp.maximum(jnp.maximum(up, rowmax), down)
    return jnp.where(hmax == x, x, 0.0)


def _nms_body(x_ref, o_ref):
    for s in range(BLK // SUB):
        x = x_ref[s * SUB : (s + 1) * SUB]
        o_ref[s * SUB : (s + 1) * SUB] = _nms_one(x)


def kernel(points):
    n, c, h, w = points.shape
    x = points.reshape(n * c, h, w)
    out = pl.pallas_call(
        _nms_body,
        grid=((n * c) // BLK,),
        in_specs=[pl.BlockSpec((BLK, h, w), lambda i: (i, 0, 0))],
        out_specs=pl.BlockSpec((BLK, h, w), lambda i: (i, 0, 0)),
        out_shape=jax.ShapeDtypeStruct((n * c, h, w), points.dtype),
        compiler_params=pltpu.CompilerParams(vmem_limit_bytes=128 * 1024 * 1024),
    )(x)
    return out.reshape(n, c, h, w)
